# baseline (speedup 1.0000x reference)
import dataclasses
import functools

import jax
import jax.numpy as jnp
from jax import lax
from jax.experimental import pallas as pl
from jax.experimental.pallas import tpu as pltpu
from jax.experimental.pallas import tpu_sc as plsc

NC = 2
NS = 16
NW = NC * NS
K = 40
BLK = 1024

_MESH = plsc.VectorSubcoreMesh(
    core_axis_name="c", subcore_axis_name="s", num_cores=NC, num_subcores=NS)

_NO_LAYOUT_CP = pltpu.CompilerParams()
if "needs_layout_passes" in pltpu.CompilerParams.__dataclass_fields__:
  _NO_LAYOUT_CP = dataclasses.replace(_NO_LAYOUT_CP, needs_layout_passes=False)


def _pad_rows(n):
  per_sub = -(-n // NS)
  per_sub = -(-per_sub // 128) * 128
  return per_sub * NS


def _fill(buf, value):
  rows_, dt = buf.shape

  @pl.loop(0, rows_)
  def _(r):
    @pl.loop(0, dt, step=16)
    def _(cc):
      buf[r, pl.ds(cc, 16)] = jnp.full((16,), value, jnp.float32)


def _zero_shared(acc_sh, buf, sid, rows_per_sub):
  zr = buf.shape[0]
  r0 = sid * rows_per_sub

  @pl.loop(0, rows_per_sub // zr)
  def _(j):
    pltpu.sync_copy(buf, acc_sh.at[pl.ds(r0 + j * zr, zr)])


def _stage_idx(all_ref, small_ref, j):
  klen = small_ref.shape[0]

  @pl.loop(0, klen - 15, step=16)
  def _(cc):
    small_ref[pl.ds(cc, 16)] = all_ref[pl.ds(j * klen + cc, 16)]

  if klen % 16:
    t0 = klen - 16
    small_ref[pl.ds(t0, 16)] = all_ref[pl.ds(j * klen + t0, 16)]


NB = 2


def _sc_segment_sum(table, src, dst):
  n, dt = table.shape
  e = src.shape[0]
  ew = e // NW
  nchunk = ew // K
  np_ = _pad_rows(n)
  rows_per_sub = np_ // NS
  assert ew * NW == e and nchunk * K == ew and nchunk % NB == 0

  @functools.partial(
      pl.kernel,
      out_type=jax.ShapeDtypeStruct((NC, np_, dt), jnp.float32),
      mesh=_MESH,
      scratch_types=[
          pltpu.VMEM_SHARED((np_, dt), jnp.float32),
          pltpu.VMEM((ew,), jnp.int32),
          pltpu.VMEM((ew,), jnp.int32),
          [pltpu.VMEM((K,), jnp.int32)] * NB,
          [pltpu.VMEM((K,), jnp.int32)] * NB,
          [pltpu.VMEM((K, dt), jnp.float32)] * NB,
          [pltpu.SemaphoreType.DMA] * NB,
      ],
  )
  def k(table_hbm, src_hbm, dst_hbm, out_hbm, acc_sh, src_all, dst_all,
        srcb, dstb, rows, gs):
    cid = lax.axis_index("c")
    sid = lax.axis_index("s")
    base = (sid * NC + cid) * ew
    pltpu.sync_copy(src_hbm.at[pl.ds(base, ew)], src_all)
    pltpu.sync_copy(dst_hbm.at[pl.ds(base, ew)], dst_all)
    _fill(rows[0], 0.0)
    _zero_shared(acc_sh, rows[0], sid, rows_per_sub)

    for b in range(NB):
      _stage_idx(src_all, srcb[b], b)
      pltpu.async_copy(table_hbm.at[srcb[b]], rows[b], gs[b])
    plsc.subcore_barrier()

    @pl.loop(0, nchunk // NB - 1)
    def _(t):
      for b in range(NB):
        j = t * NB + b
        pltpu.make_async_copy(table_hbm.at[srcb[b]], rows[b], gs[b]).wait()
        _stage_idx(dst_all, dstb[b], j)
        pltpu.sync_copy(rows[b], acc_sh.at[dstb[b]], add=True)
        _stage_idx(src_all, srcb[b], j + NB)
        pltpu.async_copy(table_hbm.at[srcb[b]], rows[b], gs[b])

    for b in range(NB):
      j = nchunk - NB + b
      pltpu.make_async_copy(table_hbm.at[srcb[b]], rows[b], gs[b]).wait()
      _stage_idx(dst_all, dstb[b], j)
      pltpu.sync_copy(rows[b], acc_sh.at[dstb[b]], add=True)

    plsc.subcore_barrier()
    r0 = sid * rows_per_sub
    pltpu.sync_copy(acc_sh.at[pl.ds(r0, rows_per_sub)],
                    out_hbm.at[cid, pl.ds(r0, rows_per_sub)])

  return k(table, src, dst)


def _sc_degree(dst, n):
  e = dst.shape[0]
  ew = e // NW
  np_ = _pad_rows(n)
  rp = np_ // 128
  assert ew * NW == e and ew % 16 == 0 and rp <= 128 and rp % 8 == 0

  @functools.partial(
      pl.kernel,
      out_type=jax.ShapeDtypeStruct((NC, rp, 128), jnp.float32),
      mesh=_MESH,
      compiler_params=_NO_LAYOUT_CP,
      scratch_types=[
          pltpu.VMEM_SHARED((rp, 128), jnp.float32),
          pltpu.VMEM((rp, 128), jnp.float32),
          pltpu.VMEM((ew,), jnp.int32),
          pltpu.VMEM((rp,), jnp.int32),
      ],
  )
  def k(dst_hbm, out_hbm, deg_sh, degv, dst_all, idb):
    cid = lax.axis_index("c")
    sid = lax.axis_index("s")
    base = (sid * NC + cid) * ew
    pltpu.sync_copy(dst_hbm.at[pl.ds(base, ew)], dst_all)
    _fill(degv, 0.0)

    @pl.when(sid < rp // 8)
    def _():
      pltpu.sync_copy(degv.at[pl.ds(0, 8)], deg_sh.at[pl.ds(sid * 8, 8)])

    for g in range(rp // 16):
      idb[pl.ds(g * 16, 16)] = jnp.arange(16, dtype=jnp.int32) + g * 16

    plsc.subcore_barrier()

    ones = jnp.ones((16,), jnp.float32)

    @pl.loop(0, ew, step=16)
    def _(i):
      d = dst_all[pl.ds(i, 16)]
      plsc.addupdate_scatter(degv, [d >> 7, d & 127], ones)

    pltpu.sync_copy(degv, deg_sh.at[idb], add=True)
    plsc.subcore_barrier()

    @pl.when(sid == 0)
    def _():
      pltpu.sync_copy(deg_sh, out_hbm.at[cid])

  return k(dst)


def _dot(a, b):
  return jnp.dot(a, b, preferred_element_type=jnp.float32,
                 precision=lax.Precision.HIGHEST)


def _mm2_body(x_ref, agg0p_ref, degp_ref, w1n_ref, w1s_ref, b1_ref,
              w2n_ref, w2s_ref, b2_ref, y2_ref, s2_ref, invd_ref):
  agg0 = agg0p_ref[0] + agg0p_ref[1]
  deg = degp_ref[0] + degp_ref[1]
  invd = 1.0 / jnp.maximum(deg, 1.0)
  x = x_ref[...]
  h1 = jnp.maximum(
      _dot(x, w1s_ref[...]) + _dot(agg0 * invd, w1n_ref[...]) + b1_ref[...],
      0.0)
  y2_ref[...] = _dot(h1, w2n_ref[...])
  s2_ref[...] = _dot(h1, w2s_ref[...]) + b2_ref[...]
  invd_ref[...] = invd


def _mm3_body(s2_ref, aggp2_ref, invd_ref, emb_ref, high_ref, low_ref, out_ref):
  agg2 = aggp2_ref[0] + aggp2_ref[1]
  h = s2_ref[...] + agg2 * invd_ref[...]
  b = h.shape[0]
  ids = (jax.lax.broadcasted_iota(jnp.int32, (b, 1), 0)
         + pl.program_id(0) * b)
  inh = jnp.any(ids == high_ref[...], axis=1, keepdims=True)
  inl = jnp.any(ids == low_ref[...], axis=1, keepdims=True)
  keep = jnp.logical_not(jnp.logical_or(inh, inl))
  out_ref[...] = (jnp.where(keep, emb_ref[...], h)
                  + jnp.where(inl, h, jnp.zeros_like(h)))


def kernel(features, embedding, W1_self, W1_neigh, b1, W2_self, W2_neigh, b2,
           edge_index, ngh_high_deg, ngh_low_deg):
  n, d = features.shape
  c = W2_self.shape[1]
  grid = (-(-n // BLK),)
  full = lambda *shape: pl.BlockSpec(shape, lambda i: (0,) * len(shape))
  rows = lambda *shape: pl.BlockSpec(shape, lambda i: (i,) + (0,) * (len(shape) - 1))
  rows3 = lambda *shape: pl.BlockSpec(shape, lambda i: (0, i, 0))

  esrc = edge_index[0]
  edst = edge_index[1]

  degp = _sc_degree(edst, n)
  degp = degp.reshape(NC, degp.shape[1] * degp.shape[2], 1)
  agg0p = _sc_segment_sum(features, esrc, edst)

  y2, s2, invd = pl.pallas_call(
      _mm2_body,
      grid=grid,
      in_specs=[rows(BLK, d), rows3(NC, BLK, d), rows3(NC, BLK, 1),
                full(d, d), full(d, d), full(1, d),
                full(d, c), full(d, c), full(1, c)],
      out_specs=[rows(BLK, c), rows(BLK, c), rows(BLK, 1)],
      out_shape=[jax.ShapeDtypeStruct((n, c), jnp.float32),
                 jax.ShapeDtypeStruct((n, c), jnp.float32),
                 jax.ShapeDtypeStruct((n, 1), jnp.float32)],
  )(features, agg0p, degp, W1_neigh, W1_self, b1.reshape(1, d),
    W2_neigh, W2_self, b2.reshape(1, c))

  agg2p = _sc_segment_sum(y2, esrc, edst)

  nh = ngh_high_deg.shape[0]
  nl = ngh_low_deg.shape[0]
  out = pl.pallas_call(
      _mm3_body,
      grid=grid,
      in_specs=[rows(BLK, c), rows3(NC, BLK, c), rows(BLK, 1), rows(BLK, c),
                full(1, nh), full(1, nl)],
      out_specs=rows(BLK, c),
      out_shape=jax.ShapeDtypeStruct((n, c), jnp.float32),
  )(s2, agg2p, invd, embedding, ngh_high_deg.reshape(1, nh),
    ngh_low_deg.reshape(1, nl))

  return out


# speedup vs baseline: 1.4663x; 1.4663x over previous
import dataclasses
import functools

import jax
import jax.numpy as jnp
from jax import lax
from jax.experimental import pallas as pl
from jax.experimental.pallas import tpu as pltpu
from jax.experimental.pallas import tpu_sc as plsc

NC = 2
NS = 16
NW = NC * NS
K = 40
BLK = 1024

_MESH = plsc.VectorSubcoreMesh(
    core_axis_name="c", subcore_axis_name="s", num_cores=NC, num_subcores=NS)

_NO_LAYOUT_CP = pltpu.CompilerParams()
if "needs_layout_passes" in pltpu.CompilerParams.__dataclass_fields__:
  _NO_LAYOUT_CP = dataclasses.replace(_NO_LAYOUT_CP, needs_layout_passes=False)


def _pad_rows(n):
  per_sub = -(-n // NS)
  per_sub = -(-per_sub // 128) * 128
  return per_sub * NS


def _fill(buf, value):
  rows_, dt = buf.shape

  @pl.loop(0, rows_)
  def _(r):
    @pl.loop(0, dt, step=16)
    def _(cc):
      buf[r, pl.ds(cc, 16)] = jnp.full((16,), value, jnp.float32)


def _zero_shared(acc_sh, buf, sid, rows_per_sub):
  zr = buf.shape[0]
  r0 = sid * rows_per_sub

  @pl.loop(0, rows_per_sub // zr)
  def _(j):
    pltpu.sync_copy(buf, acc_sh.at[pl.ds(r0 + j * zr, zr)])


def _stage_idx(all_ref, small_ref, j):
  klen = small_ref.shape[0]

  @pl.loop(0, klen - 15, step=16)
  def _(cc):
    small_ref[pl.ds(cc, 16)] = all_ref[pl.ds(j * klen + cc, 16)]

  if klen % 16:
    t0 = klen - 16
    small_ref[pl.ds(t0, 16)] = all_ref[pl.ds(j * klen + t0, 16)]


NB = 5
NSR = 2 * NB


def _sc_segment_sum(table, src, dst, with_deg):
  n, dt = table.shape
  e = src.shape[0]
  ew = e // NW
  nchunk = ew // K
  np_ = _pad_rows(n)
  rows_per_sub = np_ // NS
  rp = np_ // 128
  assert ew * NW == e and nchunk * K == ew and nchunk % NSR == 0
  assert ew % 16 == 0 and rp <= 128 and rp % 16 == 0

  out_type = [jax.ShapeDtypeStruct((NC, np_, dt), jnp.float32)]
  scratch = [
      pltpu.VMEM_SHARED((np_, dt), jnp.float32),
      pltpu.VMEM((ew,), jnp.int32),
      [pltpu.VMEM((K,), jnp.int32)] * NSR,
      [pltpu.VMEM((K,), jnp.int32)] * NB,
      [pltpu.VMEM((K, dt), jnp.float32)] * NB,
      [pltpu.SemaphoreType.DMA] * NB,
      [pltpu.SemaphoreType.DMA] * NSR,
  ]
  if with_deg:
    out_type.append(jax.ShapeDtypeStruct((NC, rp, 128), jnp.float32))
    scratch += [
        pltpu.VMEM_SHARED((rp, 128), jnp.float32),
        pltpu.VMEM((rp, 128), jnp.float32),
        pltpu.VMEM((rp,), jnp.int32),
    ]

  @functools.partial(
      pl.kernel,
      out_type=tuple(out_type) if with_deg else out_type[0],
      mesh=_MESH,
      compiler_params=_NO_LAYOUT_CP,
      scratch_types=scratch,
  )
  def k(table_hbm, src_hbm, dst_hbm, out_hbm, *rest):
    if with_deg:
      (deg_hbm, acc_sh, dst_all, srcb, dstb, rows, gs, ss,
       deg_sh, degv, idb) = rest
    else:
      acc_sh, dst_all, srcb, dstb, rows, gs, ss = rest
    cid = lax.axis_index("c")
    sid = lax.axis_index("s")
    base = (sid * NC + cid) * ew

    pltpu.sync_copy(dst_hbm.at[pl.ds(base, ew)], dst_all)
    for b in range(NSR):
      pltpu.async_copy(src_hbm.at[pl.ds(base + b * K, K)], srcb[b], ss[b])
    _fill(rows[0], 0.0)
    _zero_shared(acc_sh, rows[0], sid, rows_per_sub)
    if with_deg:
      _fill(degv, 0.0)

      @pl.when(sid < rp // 8)
      def _():
        pltpu.sync_copy(degv.at[pl.ds(0, 8)], deg_sh.at[pl.ds(sid * 8, 8)])

      for g in range(rp // 16):
        idb[pl.ds(g * 16, 16)] = jnp.arange(16, dtype=jnp.int32) + g * 16

    for b in range(NB):
      pltpu.make_async_copy(src_hbm.at[pl.ds(base, K)], srcb[b], ss[b]).wait()
      pltpu.async_copy(table_hbm.at[srcb[b]], rows[b], gs[b])

    if with_deg:
      ones = jnp.ones((16,), jnp.float32)

      @pl.loop(0, ew, step=16)
      def _(i):
        d = dst_all[pl.ds(i, 16)]
        plsc.addupdate_scatter(degv, [d >> 7, d & 127], ones)

    plsc.subcore_barrier()

    def step(j, b10, do_gather, do_src_dma):
      rb = b10 % NB
      pltpu.make_async_copy(table_hbm.at[srcb[0]], rows[rb], gs[rb]).wait()
      _stage_idx(dst_all, dstb[rb], j)
      pltpu.sync_copy(rows[rb], acc_sh.at[dstb[rb]], add=True)
      if do_gather:
        sb = (b10 + NB) % NSR
        pltpu.make_async_copy(
            src_hbm.at[pl.ds(base, K)], srcb[sb], ss[sb]).wait()
        pltpu.async_copy(table_hbm.at[srcb[sb]], rows[rb], gs[rb])
      if do_src_dma:
        pltpu.async_copy(
            src_hbm.at[pl.ds(base + (j + NSR) * K, K)], srcb[b10], ss[b10])

    @pl.loop(0, nchunk // NSR - 1)
    def _(t):
      for b10 in range(NSR):
        step(t * NSR + b10, b10, True, True)

    for b10 in range(NB):
      step(nchunk - NSR + b10, b10, True, False)
    for b10 in range(NB, NSR):
      step(nchunk - NSR + b10, b10, False, False)

    if with_deg:
      pltpu.sync_copy(degv, deg_sh.at[idb], add=True)

    plsc.subcore_barrier()
    r0 = sid * rows_per_sub
    pltpu.sync_copy(acc_sh.at[pl.ds(r0, rows_per_sub)],
                    out_hbm.at[cid, pl.ds(r0, rows_per_sub)])
    if with_deg:
      @pl.when(sid == 0)
      def _():
        pltpu.sync_copy(deg_sh, deg_hbm.at[cid])

  return k(table, src, dst)


def _dot(a, b):
  return jnp.dot(a, b, preferred_element_type=jnp.float32,
                 precision=lax.Precision.HIGHEST)


def _mm2_body(x_ref, agg0p_ref, degp_ref, w1n_ref, w1s_ref, b1_ref,
              w2n_ref, w2s_ref, b2_ref, y2_ref, s2_ref, invd_ref):
  agg0 = agg0p_ref[0] + agg0p_ref[1]
  deg = degp_ref[0] + degp_ref[1]
  invd = 1.0 / jnp.maximum(deg, 1.0)
  x = x_ref[...]
  h1 = jnp.maximum(
      _dot(x, w1s_ref[...]) + _dot(agg0 * invd, w1n_ref[...]) + b1_ref[...],
      0.0)
  y2_ref[...] = _dot(h1, w2n_ref[...])
  s2_ref[...] = _dot(h1, w2s_ref[...]) + b2_ref[...]
  invd_ref[...] = invd


def _mm3_body(s2_ref, aggp2_ref, invd_ref, emb_ref, high_ref, low_ref, out_ref):
  agg2 = aggp2_ref[0] + aggp2_ref[1]
  h = s2_ref[...] + agg2 * invd_ref[...]
  b = h.shape[0]
  ids = (jax.lax.broadcasted_iota(jnp.int32, (b, 1), 0)
         + pl.program_id(0) * b)
  inh = jnp.any(ids == high_ref[...], axis=1, keepdims=True)
  inl = jnp.any(ids == low_ref[...], axis=1, keepdims=True)
  keep = jnp.logical_not(jnp.logical_or(inh, inl))
  out_ref[...] = (jnp.where(keep, emb_ref[...], h)
                  + jnp.where(inl, h, jnp.zeros_like(h)))


def kernel(features, embedding, W1_self, W1_neigh, b1, W2_self, W2_neigh, b2,
           edge_index, ngh_high_deg, ngh_low_deg):
  n, d = features.shape
  c = W2_self.shape[1]
  grid = (-(-n // BLK),)
  full = lambda *shape: pl.BlockSpec(shape, lambda i: (0,) * len(shape))
  rows = lambda *shape: pl.BlockSpec(shape, lambda i: (i,) + (0,) * (len(shape) - 1))
  rows3 = lambda *shape: pl.BlockSpec(shape, lambda i: (0, i, 0))

  esrc = edge_index[0]
  edst = edge_index[1]

  agg0p, degp = _sc_segment_sum(features, esrc, edst, True)
  degp = degp.reshape(NC, degp.shape[1] * degp.shape[2], 1)

  y2, s2, invd = pl.pallas_call(
      _mm2_body,
      grid=grid,
      in_specs=[rows(BLK, d), rows3(NC, BLK, d), rows3(NC, BLK, 1),
                full(d, d), full(d, d), full(1, d),
                full(d, c), full(d, c), full(1, c)],
      out_specs=[rows(BLK, c), rows(BLK, c), rows(BLK, 1)],
      out_shape=[jax.ShapeDtypeStruct((n, c), jnp.float32),
                 jax.ShapeDtypeStruct((n, c), jnp.float32),
                 jax.ShapeDtypeStruct((n, 1), jnp.float32)],
  )(features, agg0p, degp, W1_neigh, W1_self, b1.reshape(1, d),
    W2_neigh, W2_self, b2.reshape(1, c))

  agg2p = _sc_segment_sum(y2, esrc, edst, False)

  nh = ngh_high_deg.shape[0]
  nl = ngh_low_deg.shape[0]
  out = pl.pallas_call(
      _mm3_body,
      grid=grid,
      in_specs=[rows(BLK, c), rows3(NC, BLK, c), rows(BLK, 1), rows(BLK, c),
                full(1, nh), full(1, nl)],
      out_specs=rows(BLK, c),
      out_shape=jax.ShapeDtypeStruct((n, c), jnp.float32),
  )(s2, agg2p, invd, embedding, ngh_high_deg.reshape(1, nh),
    ngh_low_deg.reshape(1, nl))

  return out


# speedup vs baseline: 1.4751x; 1.0060x over previous
import dataclasses
import functools

import jax
import jax.numpy as jnp
from jax import lax
from jax.experimental import pallas as pl
from jax.experimental.pallas import tpu as pltpu
from jax.experimental.pallas import tpu_sc as plsc

NC = 2
NS = 16
NW = NC * NS
K = 40
BLK = 1024

_MESH = plsc.VectorSubcoreMesh(
    core_axis_name="c", subcore_axis_name="s", num_cores=NC, num_subcores=NS)

_NO_LAYOUT_CP = pltpu.CompilerParams()
if "needs_layout_passes" in pltpu.CompilerParams.__dataclass_fields__:
  _NO_LAYOUT_CP = dataclasses.replace(_NO_LAYOUT_CP, needs_layout_passes=False)


def _pad_rows(n):
  per_sub = -(-n // NS)
  per_sub = -(-per_sub // 128) * 128
  return per_sub * NS


def _fill(buf, value):
  rows_, dt = buf.shape

  @pl.loop(0, rows_)
  def _(r):
    @pl.loop(0, dt, step=16)
    def _(cc):
      buf[r, pl.ds(cc, 16)] = jnp.full((16,), value, jnp.float32)


def _zero_shared(acc_sh, buf, sid, rows_per_sub, sem):
  zr = buf.shape[0]
  r0 = sid * rows_per_sub
  ncp = rows_per_sub // zr

  @pl.loop(0, ncp)
  def _(j):
    pltpu.async_copy(buf, acc_sh.at[pl.ds(r0 + j * zr, zr)], sem)

  @pl.loop(0, ncp)
  def _(j):
    pltpu.make_async_copy(buf, acc_sh.at[pl.ds(r0, zr)], sem).wait()


def _stage_idx(all_ref, small_ref, j):
  klen = small_ref.shape[0]

  @pl.loop(0, klen - 15, step=16)
  def _(cc):
    small_ref[pl.ds(cc, 16)] = all_ref[pl.ds(j * klen + cc, 16)]

  if klen % 16:
    t0 = klen - 16
    small_ref[pl.ds(t0, 16)] = all_ref[pl.ds(j * klen + t0, 16)]


NB = 5
NSR = 2 * NB


def _sc_segment_sum(table, src, dst, with_deg):
  n, dt = table.shape
  e = src.shape[0]
  ew = e // NW
  nchunk = ew // K
  np_ = _pad_rows(n)
  rows_per_sub = np_ // NS
  rp = np_ // 128
  assert ew * NW == e and nchunk * K == ew and nchunk % NSR == 0
  assert ew % 16 == 0 and rp <= 128 and rp % 16 == 0

  out_type = [jax.ShapeDtypeStruct((NC, np_, dt), jnp.float32)]
  scratch = [
      pltpu.VMEM_SHARED((np_, dt), jnp.float32),
      pltpu.VMEM((ew,), jnp.int32),
      [pltpu.VMEM((K,), jnp.int32)] * NSR,
      [pltpu.VMEM((K,), jnp.int32)] * NB,
      [pltpu.VMEM((K, dt), jnp.float32)] * NB,
      [pltpu.SemaphoreType.DMA] * NB,
      [pltpu.SemaphoreType.DMA] * NSR,
      pltpu.SemaphoreType.DMA,
  ]
  if with_deg:
    out_type.append(jax.ShapeDtypeStruct((NC, rp, 128), jnp.float32))
    scratch += [
        pltpu.VMEM_SHARED((rp, 128), jnp.float32),
        pltpu.VMEM((rp, 128), jnp.float32),
        pltpu.VMEM((rp,), jnp.int32),
    ]

  @functools.partial(
      pl.kernel,
      out_type=tuple(out_type) if with_deg else out_type[0],
      mesh=_MESH,
      compiler_params=_NO_LAYOUT_CP,
      scratch_types=scratch,
  )
  def k(table_hbm, src_hbm, dst_hbm, out_hbm, *rest):
    if with_deg:
      (deg_hbm, acc_sh, dst_all, srcb, dstb, rows, gs, ss, zs,
       deg_sh, degv, idb) = rest
    else:
      acc_sh, dst_all, srcb, dstb, rows, gs, ss, zs = rest
    cid = lax.axis_index("c")
    sid = lax.axis_index("s")
    base = (sid * NC + cid) * ew

    pltpu.async_copy(dst_hbm.at[pl.ds(base, ew)], dst_all, zs)
    for b in range(NSR):
      pltpu.async_copy(src_hbm.at[pl.ds(base + b * K, K)], srcb[b], ss[b])
    _fill(rows[0], 0.0)
    if with_deg:
      _fill(degv, 0.0)
      for g in range(rp // 16):
        idb[pl.ds(g * 16, 16)] = jnp.arange(16, dtype=jnp.int32) + g * 16
    pltpu.make_async_copy(dst_hbm.at[pl.ds(base, ew)], dst_all, zs).wait()
    _zero_shared(acc_sh, rows[0], sid, rows_per_sub, zs)
    if with_deg:
      @pl.when(sid < rp // 8)
      def _():
        pltpu.sync_copy(degv.at[pl.ds(0, 8)], deg_sh.at[pl.ds(sid * 8, 8)])

    for b in range(NB):
      pltpu.make_async_copy(src_hbm.at[pl.ds(base, K)], srcb[b], ss[b]).wait()
      pltpu.async_copy(table_hbm.at[srcb[b]], rows[b], gs[b])

    if with_deg:
      ones = jnp.ones((16,), jnp.float32)

      @pl.loop(0, ew, step=16)
      def _(i):
        d = dst_all[pl.ds(i, 16)]
        plsc.addupdate_scatter(degv, [d >> 7, d & 127], ones)

    plsc.subcore_barrier()

    def step(j, b10, do_gather, do_src_dma):
      rb = b10 % NB
      pltpu.make_async_copy(table_hbm.at[srcb[0]], rows[rb], gs[rb]).wait()
      _stage_idx(dst_all, dstb[rb], j)
      pltpu.sync_copy(rows[rb], acc_sh.at[dstb[rb]], add=True)
      if do_gather:
        sb = (b10 + NB) % NSR
        pltpu.make_async_copy(
            src_hbm.at[pl.ds(base, K)], srcb[sb], ss[sb]).wait()
        pltpu.async_copy(table_hbm.at[srcb[sb]], rows[rb], gs[rb])
      if do_src_dma:
        pltpu.async_copy(
            src_hbm.at[pl.ds(base + (j + NSR) * K, K)], srcb[b10], ss[b10])

    @pl.loop(0, nchunk // NSR - 1)
    def _(t):
      for b10 in range(NSR):
        step(t * NSR + b10, b10, True, True)

    for b10 in range(NB):
      step(nchunk - NSR + b10, b10, True, False)
    for b10 in range(NB, NSR):
      step(nchunk - NSR + b10, b10, False, False)

    if with_deg:
      pltpu.sync_copy(degv, deg_sh.at[idb], add=True)

    plsc.subcore_barrier()
    r0 = sid * rows_per_sub
    pltpu.sync_copy(acc_sh.at[pl.ds(r0, rows_per_sub)],
                    out_hbm.at[cid, pl.ds(r0, rows_per_sub)])
    if with_deg:
      @pl.when(sid == 0)
      def _():
        pltpu.sync_copy(deg_sh, deg_hbm.at[cid])

  return k(table, src, dst)


def _dot(a, b):
  return jnp.dot(a, b, preferred_element_type=jnp.float32,
                 precision=lax.Precision.HIGHEST)


def _mm2_body(x_ref, agg0p_ref, degp_ref, w1n_ref, w1s_ref, b1_ref,
              w2n_ref, w2s_ref, b2_ref, y2_ref, s2_ref, invd_ref):
  agg0 = agg0p_ref[0] + agg0p_ref[1]
  deg = degp_ref[0] + degp_ref[1]
  invd = 1.0 / jnp.maximum(deg, 1.0)
  x = x_ref[...]
  h1 = jnp.maximum(
      _dot(x, w1s_ref[...]) + _dot(agg0 * invd, w1n_ref[...]) + b1_ref[...],
      0.0)
  y2_ref[...] = _dot(h1, w2n_ref[...])
  s2_ref[...] = _dot(h1, w2s_ref[...]) + b2_ref[...]
  invd_ref[...] = invd


def _mm3_body(s2_ref, aggp2_ref, invd_ref, emb_ref, high_ref, low_ref, out_ref):
  agg2 = aggp2_ref[0] + aggp2_ref[1]
  h = s2_ref[...] + agg2 * invd_ref[...]
  b = h.shape[0]
  ids = (jax.lax.broadcasted_iota(jnp.int32, (b, 1), 0)
         + pl.program_id(0) * b)
  inh = jnp.any(ids == high_ref[...], axis=1, keepdims=True)
  inl = jnp.any(ids == low_ref[...], axis=1, keepdims=True)
  keep = jnp.logical_not(jnp.logical_or(inh, inl))
  out_ref[...] = (jnp.where(keep, emb_ref[...], h)
                  + jnp.where(inl, h, jnp.zeros_like(h)))


def kernel(features, embedding, W1_self, W1_neigh, b1, W2_self, W2_neigh, b2,
           edge_index, ngh_high_deg, ngh_low_deg):
  n, d = features.shape
  c = W2_self.shape[1]
  grid = (-(-n // BLK),)
  full = lambda *shape: pl.BlockSpec(shape, lambda i: (0,) * len(shape))
  rows = lambda *shape: pl.BlockSpec(shape, lambda i: (i,) + (0,) * (len(shape) - 1))
  rows3 = lambda *shape: pl.BlockSpec(shape, lambda i: (0, i, 0))

  esrc = edge_index[0]
  edst = edge_index[1]

  agg0p, degp = _sc_segment_sum(features, esrc, edst, True)
  degp = degp.reshape(NC, degp.shape[1] * degp.shape[2], 1)

  y2, s2, invd = pl.pallas_call(
      _mm2_body,
      grid=grid,
      in_specs=[rows(BLK, d), rows3(NC, BLK, d), rows3(NC, BLK, 1),
                full(d, d), full(d, d), full(1, d),
                full(d, c), full(d, c), full(1, c)],
      out_specs=[rows(BLK, c), rows(BLK, c), rows(BLK, 1)],
      out_shape=[jax.ShapeDtypeStruct((n, c), jnp.float32),
                 jax.ShapeDtypeStruct((n, c), jnp.float32),
                 jax.ShapeDtypeStruct((n, 1), jnp.float32)],
  )(features, agg0p, degp, W1_neigh, W1_self, b1.reshape(1, d),
    W2_neigh, W2_self, b2.reshape(1, c))

  agg2p = _sc_segment_sum(y2, esrc, edst, False)

  nh = ngh_high_deg.shape[0]
  nl = ngh_low_deg.shape[0]
  out = pl.pallas_call(
      _mm3_body,
      grid=grid,
      in_specs=[rows(BLK, c), rows3(NC, BLK, c), rows(BLK, 1), rows(BLK, c),
                full(1, nh), full(1, nl)],
      out_specs=rows(BLK, c),
      out_shape=jax.ShapeDtypeStruct((n, c), jnp.float32),
  )(s2, agg2p, invd, embedding, ngh_high_deg.reshape(1, nh),
    ngh_low_deg.reshape(1, nl))

  return out


# speedup vs baseline: 1.5903x; 1.0781x over previous
import dataclasses
import functools

import jax
import jax.numpy as jnp
from jax import lax
from jax.experimental import pallas as pl
from jax.experimental.pallas import tpu as pltpu
from jax.experimental.pallas import tpu_sc as plsc

NC = 2
NS = 16
NW = NC * NS
K = 40
BLK = 1024

_MESH = plsc.VectorSubcoreMesh(
    core_axis_name="c", subcore_axis_name="s", num_cores=NC, num_subcores=NS)

_NO_LAYOUT_CP = pltpu.CompilerParams()
if "needs_layout_passes" in pltpu.CompilerParams.__dataclass_fields__:
  _NO_LAYOUT_CP = dataclasses.replace(_NO_LAYOUT_CP, needs_layout_passes=False)


def _pad_rows(n):
  per_sub = -(-n // NS)
  per_sub = -(-per_sub // 128) * 128
  return per_sub * NS


def _fill(buf, value):
  rows_, dt = buf.shape

  @pl.loop(0, rows_)
  def _(r):
    @pl.loop(0, dt, step=16)
    def _(cc):
      buf[r, pl.ds(cc, 16)] = jnp.full((16,), value, jnp.float32)


def _zero_shared(acc_sh, buf, sid, rows_per_sub, sem):
  zr = buf.shape[0]
  r0 = sid * rows_per_sub
  ncp = rows_per_sub // zr

  @pl.loop(0, ncp)
  def _(j):
    pltpu.async_copy(buf, acc_sh.at[pl.ds(r0 + j * zr, zr)], sem)

  @pl.loop(0, ncp)
  def _(j):
    pltpu.make_async_copy(buf, acc_sh.at[pl.ds(r0, zr)], sem).wait()


def _stage_idx(all_ref, small_ref, j):
  klen = small_ref.shape[0]

  @pl.loop(0, klen - 15, step=16)
  def _(cc):
    small_ref[pl.ds(cc, 16)] = all_ref[pl.ds(j * klen + cc, 16)]

  if klen % 16:
    t0 = klen - 16
    small_ref[pl.ds(t0, 16)] = all_ref[pl.ds(j * klen + t0, 16)]


NB = 5
NSR = 2 * NB


def _sc_segment_sum(table, src, dst, with_deg):
  n, dt = table.shape
  e = src.shape[0]
  ew = e // NW
  nchunk = ew // K
  np_ = _pad_rows(n)
  rows_per_sub = np_ // NS
  rp = np_ // 128
  assert ew * NW == e and nchunk * K == ew and nchunk % NSR == 0
  assert ew % 16 == 0 and rp <= 128 and rp % 16 == 0

  out_type = [jax.ShapeDtypeStruct((NC, np_, dt), jnp.float32)]
  scratch = [
      pltpu.VMEM_SHARED((np_, dt), jnp.float32),
      pltpu.VMEM((ew,), jnp.int32),
      [pltpu.VMEM((K,), jnp.int32)] * NSR,
      [pltpu.VMEM((K,), jnp.int32)] * NB,
      [pltpu.VMEM((K, dt), jnp.float32)] * NB,
      [pltpu.SemaphoreType.DMA] * NB,
      [pltpu.SemaphoreType.DMA] * NSR,
      pltpu.SemaphoreType.DMA,
  ]
  if with_deg:
    out_type.append(jax.ShapeDtypeStruct((NC, rp, 128), jnp.float32))
    scratch += [
        pltpu.VMEM_SHARED((rp, 128), jnp.float32),
        pltpu.VMEM((rp, 128), jnp.float32),
        pltpu.VMEM((rp,), jnp.int32),
    ]

  @functools.partial(
      pl.kernel,
      out_type=tuple(out_type) if with_deg else out_type[0],
      mesh=_MESH,
      compiler_params=_NO_LAYOUT_CP,
      scratch_types=scratch,
  )
  def k(table_hbm, src_hbm, dst_hbm, out_hbm, *rest):
    if with_deg:
      (deg_hbm, acc_sh, dst_all, srcb, dstb, rows, gs, ss, zs,
       deg_sh, degv, idb) = rest
    else:
      acc_sh, dst_all, srcb, dstb, rows, gs, ss, zs = rest
    cid = lax.axis_index("c")
    sid = lax.axis_index("s")
    base = (sid * NC + cid) * ew

    pltpu.async_copy(dst_hbm.at[pl.ds(base, ew)], dst_all, zs)
    for b in range(NSR):
      pltpu.async_copy(src_hbm.at[pl.ds(base + b * K, K)], srcb[b], ss[b])
    _fill(rows[0], 0.0)
    if with_deg:
      _fill(degv, 0.0)
      for g in range(rp // 16):
        idb[pl.ds(g * 16, 16)] = jnp.arange(16, dtype=jnp.int32) + g * 16
    pltpu.make_async_copy(dst_hbm.at[pl.ds(base, ew)], dst_all, zs).wait()
    _zero_shared(acc_sh, rows[0], sid, rows_per_sub, zs)
    if with_deg:
      @pl.when(sid < rp // 8)
      def _():
        pltpu.sync_copy(degv.at[pl.ds(0, 8)], deg_sh.at[pl.ds(sid * 8, 8)])

    for b in range(NB):
      pltpu.make_async_copy(src_hbm.at[pl.ds(base, K)], srcb[b], ss[b]).wait()
      pltpu.async_copy(table_hbm.at[srcb[b]], rows[b], gs[b])

    if with_deg:
      ones = jnp.ones((16,), jnp.float32)

      @pl.loop(0, ew, step=16)
      def _(i):
        d = dst_all[pl.ds(i, 16)]
        plsc.addupdate_scatter(degv, [d >> 7, d & 127], ones)

    plsc.subcore_barrier()

    def step(j, b10, do_gather, do_src_dma):
      rb = b10 % NB
      pltpu.make_async_copy(table_hbm.at[srcb[0]], rows[rb], gs[rb]).wait()
      _stage_idx(dst_all, dstb[rb], j)
      pltpu.sync_copy(rows[rb], acc_sh.at[dstb[rb]], add=True)
      if do_gather:
        sb = (b10 + NB) % NSR
        pltpu.make_async_copy(
            src_hbm.at[pl.ds(base, K)], srcb[sb], ss[sb]).wait()
        pltpu.async_copy(table_hbm.at[srcb[sb]], rows[rb], gs[rb])
      if do_src_dma:
        pltpu.async_copy(
            src_hbm.at[pl.ds(base + (j + NSR) * K, K)], srcb[b10], ss[b10])

    @pl.loop(0, nchunk // NSR - 1)
    def _(t):
      for b10 in range(NSR):
        step(t * NSR + b10, b10, True, True)

    for b10 in range(NB):
      step(nchunk - NSR + b10, b10, True, False)
    for b10 in range(NB, NSR):
      step(nchunk - NSR + b10, b10, False, False)

    if with_deg:
      pltpu.sync_copy(degv, deg_sh.at[idb], add=True)

    plsc.subcore_barrier()
    r0 = sid * rows_per_sub
    pltpu.sync_copy(acc_sh.at[pl.ds(r0, rows_per_sub)],
                    out_hbm.at[cid, pl.ds(r0, rows_per_sub)])
    if with_deg:
      @pl.when(sid == 0)
      def _():
        pltpu.sync_copy(deg_sh, deg_hbm.at[cid])

  return k(table, src, dst)


def _dot(a, b):
  return jnp.dot(a, b, preferred_element_type=jnp.float32,
                 precision=lax.Precision.DEFAULT)


def _mm2_body(x_ref, agg0p_ref, degp_ref, w1n_ref, w1s_ref, b1_ref,
              w2n_ref, w2s_ref, b2_ref, y2_ref, s2_ref, invd_ref):
  agg0 = agg0p_ref[0] + agg0p_ref[1]
  deg = degp_ref[0] + degp_ref[1]
  invd = 1.0 / jnp.maximum(deg, 1.0)
  x = x_ref[...]
  h1 = jnp.maximum(
      _dot(x, w1s_ref[...]) + _dot(agg0 * invd, w1n_ref[...]) + b1_ref[...],
      0.0)
  y2_ref[...] = _dot(h1, w2n_ref[...])
  s2_ref[...] = _dot(h1, w2s_ref[...]) + b2_ref[...]
  invd_ref[...] = invd


def _mm3_body(s2_ref, aggp2_ref, invd_ref, emb_ref, high_ref, low_ref, out_ref):
  agg2 = aggp2_ref[0] + aggp2_ref[1]
  h = s2_ref[...] + agg2 * invd_ref[...]
  b = h.shape[0]
  ids = (jax.lax.broadcasted_iota(jnp.int32, (b, 1), 0)
         + pl.program_id(0) * b)
  inh = jnp.any(ids == high_ref[...], axis=1, keepdims=True)
  inl = jnp.any(ids == low_ref[...], axis=1, keepdims=True)
  keep = jnp.logical_not(jnp.logical_or(inh, inl))
  out_ref[...] = (jnp.where(keep, emb_ref[...], h)
                  + jnp.where(inl, h, jnp.zeros_like(h)))


def kernel(features, embedding, W1_self, W1_neigh, b1, W2_self, W2_neigh, b2,
           edge_index, ngh_high_deg, ngh_low_deg):
  n, d = features.shape
  c = W2_self.shape[1]
  grid = (-(-n // BLK),)
  full = lambda *shape: pl.BlockSpec(shape, lambda i: (0,) * len(shape))
  rows = lambda *shape: pl.BlockSpec(shape, lambda i: (i,) + (0,) * (len(shape) - 1))
  rows3 = lambda *shape: pl.BlockSpec(shape, lambda i: (0, i, 0))

  esrc = edge_index[0]
  edst = edge_index[1]

  agg0p, degp = _sc_segment_sum(features, esrc, edst, True)
  degp = degp.reshape(NC, degp.shape[1] * degp.shape[2], 1)

  y2, s2, invd = pl.pallas_call(
      _mm2_body,
      grid=grid,
      in_specs=[rows(BLK, d), rows3(NC, BLK, d), rows3(NC, BLK, 1),
                full(d, d), full(d, d), full(1, d),
                full(d, c), full(d, c), full(1, c)],
      out_specs=[rows(BLK, c), rows(BLK, c), rows(BLK, 1)],
      out_shape=[jax.ShapeDtypeStruct((n, c), jnp.float32),
                 jax.ShapeDtypeStruct((n, c), jnp.float32),
                 jax.ShapeDtypeStruct((n, 1), jnp.float32)],
  )(features, agg0p, degp, W1_neigh, W1_self, b1.reshape(1, d),
    W2_neigh, W2_self, b2.reshape(1, c))

  agg2p = _sc_segment_sum(y2, esrc, edst, False)

  nh = ngh_high_deg.shape[0]
  nl = ngh_low_deg.shape[0]
  out = pl.pallas_call(
      _mm3_body,
      grid=grid,
      in_specs=[rows(BLK, c), rows3(NC, BLK, c), rows(BLK, 1), rows(BLK, c),
                full(1, nh), full(1, nl)],
      out_specs=rows(BLK, c),
      out_shape=jax.ShapeDtypeStruct((n, c), jnp.float32),
  )(s2, agg2p, invd, embedding, ngh_high_deg.reshape(1, nh),
    ngh_low_deg.reshape(1, nl))

  return out
